# Initial kernel scaffold; baseline (speedup 1.0000x reference)
import jax
import jax.numpy as jnp
from jax.experimental import pallas as pl


def kernel(inputs, segment_ids, lengths, W1, b1, W2, b2, W3, b3, Wr, br, W_k, W_q):
    raise NotImplementedError("write your pallas kernel here")


# speedup vs baseline: 8.7095x; 8.7095x over previous
import math

import jax
import jax.numpy as jnp
from jax import lax
from jax.experimental import pallas as pl
from jax.experimental.pallas import tpu as pltpu

_N = 32768
_B = 16
_D = 128
_DP = 64
_H = 4
_BN = 4096
_G = _N // _BN
_SCALE = 1.0 / math.sqrt(float(_DP))


def _body(x_ref, seg_ref, wk_ref, wqbd_ref, out_ref, e_ref, stats_ref):
    p = pl.program_id(0)
    g = pl.program_id(1)
    hi = lax.Precision.HIGHEST
    ohT = (lax.broadcasted_iota(jnp.int32, (_B, _BN), 0) == seg_ref[...]).astype(jnp.float32)

    @pl.when(p == 0)
    def _phase0():
        u = lax.dot_general(wk_ref[...], wqbd_ref[...], (((1,), (0,)), ((), ())),
                            precision=hi)
        t = lax.dot_general(x_ref[...], u, (((1,), (0,)), ((), ())),
                            precision=hi) * _SCALE
        e = jnp.exp(jnp.minimum(t, 50.0))
        e_ref[pl.ds(g * _BN, _BN), :] = e
        part = lax.dot_general(ohT, e, (((1,), (0,)), ((), ())), precision=hi)
        out_ref[...] = e

        @pl.when(g == 0)
        def _init():
            stats_ref[...] = part

        @pl.when(g != 0)
        def _acc():
            stats_ref[...] = stats_ref[...] + part

    @pl.when(p == 1)
    def _phase1():
        e = e_ref[pl.ds(g * _BN, _BN), :]
        recip = 1.0 / jnp.maximum(stats_ref[...], 1e-30)
        rg = lax.dot_general(ohT, recip, (((0,), (0,)), ((), ())), precision=hi)
        out_ref[...] = e * rg


def _make_call(interpret=False):
    return pl.pallas_call(
        _body,
        grid=(2, _G),
        in_specs=[
            pl.BlockSpec((_BN, _D), lambda p, g: (g * (1 - p), 0)),
            pl.BlockSpec((1, _BN), lambda p, g: (0, g)),
            pl.BlockSpec((_D, _H * _DP), lambda p, g: (0, 0)),
            pl.BlockSpec((_H * _DP, _H), lambda p, g: (0, 0)),
        ],
        out_specs=pl.BlockSpec((_BN, _H), lambda p, g: (g, 0)),
        out_shape=jax.ShapeDtypeStruct((_N, _H), jnp.float32),
        scratch_shapes=[
            pltpu.VMEM((_N, _H), jnp.float32),
            pltpu.VMEM((_B, _H), jnp.float32),
        ],
        interpret=interpret,
    )


def kernel(inputs, segment_ids, lengths, W1, b1, W2, b2, W3, b3, Wr, br, W_k, W_q):
    seg_row = segment_ids.astype(jnp.int32).reshape(1, _N)
    wk_top = W_k[:_D, :]
    eye = jnp.eye(_H, dtype=jnp.float32)
    wqbd = (W_q[:, :, None] * eye[:, None, :]).reshape(_H * _DP, _H)

    out = _make_call()(inputs, seg_row, wk_top, wqbd)
    return jnp.transpose(out).reshape(_H, _N, 1)


# speedup vs baseline: 18.0326x; 2.0705x over previous
import math

import jax
import jax.numpy as jnp
from jax import lax
from jax.experimental import pallas as pl
from jax.experimental.pallas import tpu as pltpu

_N = 32768
_B = 16
_D = 128
_DP = 64
_H = 4
_BN = 4096
_G = _N // _BN
_SCALE = 1.0 / math.sqrt(float(_DP))


def _body(x_ref, seg_ref, wk_ref, wqbd_ref, out_ref, e_ref, stats_ref):
    p = pl.program_id(0)
    g = pl.program_id(1)
    hi = lax.Precision.HIGHEST
    seg = jnp.broadcast_to(seg_ref[...], (_H, _BN))

    @pl.when(p == 0)
    def _phase0():
        u = lax.dot_general(wk_ref[...], wqbd_ref[...], (((1,), (0,)), ((), ())),
                            precision=hi)
        t = lax.dot_general(x_ref[...], u, (((1,), (0,)), ((), ())),
                            precision=hi) * _SCALE
        e = jnp.exp(jnp.minimum(jnp.transpose(t), 50.0))
        e_ref[:, pl.ds(g * _BN, _BN)] = e
        out_ref[...] = e
        parts = []
        for s in range(_B):
            m = seg == s
            parts.append(jnp.sum(jnp.where(m, e, 0.0), axis=1, keepdims=True))
        part = jnp.concatenate(parts, axis=1)

        @pl.when(g == 0)
        def _init():
            stats_ref[...] = part

        @pl.when(g != 0)
        def _acc():
            stats_ref[...] = stats_ref[...] + part

    @pl.when(p == 1)
    def _phase1():
        e = e_ref[:, pl.ds(g * _BN, _BN)]
        recip = 1.0 / jnp.maximum(stats_ref[...], 1e-30)
        rg = jnp.broadcast_to(recip[:, 0:1], (_H, _BN))
        for s in range(1, _B):
            rg = jnp.where(seg == s, jnp.broadcast_to(recip[:, s:s + 1], (_H, _BN)), rg)
        out_ref[...] = e * rg


def _make_call(interpret=False):
    return pl.pallas_call(
        _body,
        grid=(2, _G),
        in_specs=[
            pl.BlockSpec((_BN, _D), lambda p, g: (g * (1 - p), 0)),
            pl.BlockSpec((1, _BN), lambda p, g: (0, g)),
            pl.BlockSpec((_D, _H * _DP), lambda p, g: (0, 0)),
            pl.BlockSpec((_H * _DP, _H), lambda p, g: (0, 0)),
        ],
        out_specs=pl.BlockSpec((_H, _BN), lambda p, g: (0, g)),
        out_shape=jax.ShapeDtypeStruct((_H, _N), jnp.float32),
        scratch_shapes=[
            pltpu.VMEM((_H, _N), jnp.float32),
            pltpu.VMEM((_H, _B), jnp.float32),
        ],
        interpret=interpret,
    )


def kernel(inputs, segment_ids, lengths, W1, b1, W2, b2, W3, b3, Wr, br, W_k, W_q):
    seg_row = segment_ids.astype(jnp.int32).reshape(1, _N)
    wk_top = W_k[:_D, :]
    eye = jnp.eye(_H, dtype=jnp.float32)
    wqbd = (W_q[:, :, None] * eye[:, None, :]).reshape(_H * _DP, _H)

    out = _make_call()(inputs, seg_row, wk_top, wqbd)
    return out.reshape(_H, _N, 1)


# speedup vs baseline: 25.7417x; 1.4275x over previous
import math

import jax
import jax.numpy as jnp
from jax import lax
from jax.experimental import pallas as pl
from jax.experimental.pallas import tpu as pltpu

_N = 32768
_B = 16
_D = 128
_DP = 64
_H = 4
_BN = 4096
_G = _N // _BN
_SCALE = 1.0 / math.sqrt(float(_DP))


def _body(x_ref, seg_ref, wk_ref, wqbd_ref, out_ref, e_ref, stats_ref, u_ref):
    p = pl.program_id(0)
    g = pl.program_id(1)
    seg = jnp.broadcast_to(seg_ref[...], (_H, _BN))

    @pl.when((p == 0) & (g == 0))
    def _fold_u():
        u_ref[...] = lax.dot_general(wk_ref[...], wqbd_ref[...],
                                     (((1,), (0,)), ((), ())),
                                     precision=lax.Precision.HIGHEST)

    @pl.when(p == 0)
    def _phase0():
        t = lax.dot_general(x_ref[...], u_ref[...], (((1,), (0,)), ((), ())))
        t = t * _SCALE
        e0 = jnp.exp(jnp.minimum(t, 50.0))
        e = jnp.transpose(e0)
        e_ref[:, pl.ds(g * _BN, _BN)] = e
        out_ref[...] = e
        ohT = (lax.broadcasted_iota(jnp.int32, (_B, _BN), 0)
               == seg_ref[...]).astype(jnp.float32)
        part = jnp.transpose(
            lax.dot_general(ohT, e0, (((1,), (0,)), ((), ()))))

        @pl.when(g == 0)
        def _init():
            stats_ref[...] = part

        @pl.when(g != 0)
        def _acc():
            stats_ref[...] = stats_ref[...] + part

    @pl.when(p == 1)
    def _phase1():
        e = e_ref[:, pl.ds(g * _BN, _BN)]
        recip = 1.0 / jnp.maximum(stats_ref[...], 1e-30)
        rg = jnp.broadcast_to(recip[:, 0:1], (_H, _BN))
        for s in range(1, _B):
            rg = jnp.where(seg == s, jnp.broadcast_to(recip[:, s:s + 1], (_H, _BN)), rg)
        out_ref[...] = e * rg


def _make_call(interpret=False):
    return pl.pallas_call(
        _body,
        grid=(2, _G),
        in_specs=[
            pl.BlockSpec((_BN, _D), lambda p, g: (g * (1 - p), 0)),
            pl.BlockSpec((1, _BN), lambda p, g: (0, g)),
            pl.BlockSpec((_D, _H * _DP), lambda p, g: (0, 0)),
            pl.BlockSpec((_H * _DP, _H), lambda p, g: (0, 0)),
        ],
        out_specs=pl.BlockSpec((_H, _BN), lambda p, g: (0, g)),
        out_shape=jax.ShapeDtypeStruct((_H, _N), jnp.float32),
        scratch_shapes=[
            pltpu.VMEM((_H, _N), jnp.float32),
            pltpu.VMEM((_H, _B), jnp.float32),
            pltpu.VMEM((_D, _H), jnp.float32),
        ],
        interpret=interpret,
    )


def kernel(inputs, segment_ids, lengths, W1, b1, W2, b2, W3, b3, Wr, br, W_k, W_q):
    seg_row = segment_ids.astype(jnp.int32).reshape(1, _N)
    wk_top = W_k[:_D, :]
    eye = jnp.eye(_H, dtype=jnp.float32)
    wqbd = (W_q[:, :, None] * eye[:, None, :]).reshape(_H * _DP, _H)

    out = _make_call()(inputs, seg_row, wk_top, wqbd)
    return out.reshape(_H, _N, 1)
